# Initial kernel scaffold; baseline (speedup 1.0000x reference)
import jax
import jax.numpy as jnp
from jax.experimental import pallas as pl


def kernel(x, edge_index, graph_input, batch, params):
    raise NotImplementedError("write your pallas kernel here")


# speedup vs baseline: 6.3605x; 6.3605x over previous
import functools
import math

import jax
import jax.numpy as jnp
from jax import lax
from jax.experimental import pallas as pl
from jax.experimental.pallas import tpu as pltpu
from jax.experimental.pallas import tpu_sc as plsc

_N = 10000
_E = 160000
_F = 128
_HID = 128
_GF = 16
_NC = 10
_B = 64
_K = 4
_BN_INV = 1.0 / math.sqrt(1.0 + 1e-5)
_GN_EPS = 1e-5

_R = 256
_NPAD = 10240
_NBLK = _NPAD // _R
_BE = 2000
_BN = 1000
_INF = jnp.inf


def _onehot(batf, n):
    io = lax.broadcasted_iota(jnp.int32, (n, 128), 1).astype(jnp.float32)
    return jnp.where(batf == io, 1.0, 0.0)


def _bounds_body(batf_ref, lo_ref, hi_ref):
    batf = batf_ref[...]
    oh = _onehot(batf, _NPAD)
    ones_col = jnp.ones((_NPAD, 1), jnp.float32)
    cnt_col = lax.dot_general(oh, ones_col, (((0,), (0,)), ((), ())))
    li = lax.broadcasted_iota(jnp.int32, (128, 128), 0)
    lj = lax.broadcasted_iota(jnp.int32, (128, 128), 1)
    lower = jnp.where(li < lj, 1.0, 0.0)
    starts_col = lax.dot_general(lower, cnt_col, (((0,), (0,)), ((), ())))
    ends_col = starts_col + cnt_col
    rio = lax.broadcasted_iota(jnp.int32, (_NBLK, _NPAD), 1)
    bio = lax.broadcasted_iota(jnp.int32, (_NBLK, _NPAD), 0)
    sel_first = jnp.where(rio == bio * _R, 1.0, 0.0)
    sel_last = jnp.where(rio == bio * _R + (_R - 1), 1.0, 0.0)
    b_first = lax.dot_general(sel_first, batf, (((1,), (0,)), ((), ())))
    b_last = lax.dot_general(sel_last, batf, (((1,), (0,)), ((), ())))
    oh_f = _onehot(b_first, _NBLK)
    oh_l = _onehot(b_last, _NBLK)
    lo = lax.dot_general(oh_f, starts_col, (((1,), (0,)), ((), ())))
    hi = lax.dot_general(oh_l, ends_col, (((1,), (0,)), ((), ())))
    lo_ref[...] = lo.astype(jnp.int32)
    hi_ref[...] = hi.astype(jnp.int32)


def _block_bounds(batf_pad):
    return pl.pallas_call(
        _bounds_body,
        out_shape=[jax.ShapeDtypeStruct((_NBLK, 1), jnp.int32),
                   jax.ShapeDtypeStruct((_NBLK, 1), jnp.int32)],
    )(batf_pad)


def _gn_math(t, batf, w, b, ms):
    oh = _onehot(batf, _N)
    ones_col = jnp.ones((_N, 1), jnp.float32)
    cnt_col = lax.dot_general(oh, ones_col, (((0,), (0,)), ((), ())))
    cnt_col = jnp.maximum(cnt_col, 1.0)
    seg_sum = lax.dot_general(oh, t, (((0,), (0,)), ((), ())))
    mean = seg_sum / cnt_col
    mean_b = jnp.dot(oh, mean)
    out = t - mean_b * ms
    var_sum = lax.dot_general(oh, out * out, (((0,), (0,)), ((), ())))
    std = jnp.sqrt(var_sum / cnt_col + _GN_EPS)
    std_b = jnp.dot(oh, std)
    return w * out / std_b + b


def _gn_plain_body(t_ref, batf_ref, w_ref, b_ref, ms_ref, o_ref):
    o_ref[...] = _gn_math(t_ref[...], batf_ref[...], w_ref[...], b_ref[...],
                          ms_ref[...])


def _gn_plain(t, batf, p):
    return pl.pallas_call(
        _gn_plain_body,
        out_shape=jax.ShapeDtypeStruct((_N, _HID), jnp.float32),
    )(t, batf, p['w'].reshape(1, -1), p['b'].reshape(1, -1),
      p['ms'].reshape(1, -1))


def _gn_conv_body(agg_ref, xp_ref, wst_ref, bs_ref, batf_ref, w_ref, b_ref,
                  ms_ref, o_ref):
    agg = agg_ref[...]
    agg = jnp.where(agg == -_INF, 0.0, agg)
    short = (jnp.dot(xp_ref[...], wst_ref[...]) + bs_ref[...]) * _BN_INV
    o_ref[...] = _gn_math(agg + short, batf_ref[...], w_ref[...], b_ref[...],
                          ms_ref[...])


def _gn_conv(agg, xprev, cp, batf, gp):
    return pl.pallas_call(
        _gn_conv_body,
        out_shape=jax.ShapeDtypeStruct((_N, _HID), jnp.float32),
    )(agg, xprev, cp['Ws'].T, cp['bs'].reshape(1, -1), batf,
      gp['w'].reshape(1, -1), gp['b'].reshape(1, -1), gp['ms'].reshape(1, -1))


def _gather_rows(table, idx):
    m = idx.shape[0]
    d = table.shape[1]
    w = 128
    mesh = plsc.VectorSubcoreMesh(core_axis_name="core",
                                  subcore_axis_name="subcore")

    @functools.partial(
        pl.kernel,
        out_type=jax.ShapeDtypeStruct((m, d), table.dtype),
        mesh=mesh)
    def k(x_hbm, i_hbm, o_hbm):
        def body(i_vmem, o_vmem):
            pltpu.sync_copy(x_hbm.at[i_vmem.at[0]], o_vmem)

        pltpu.emit_pipeline(
            body,
            grid=(m // w,),
            in_specs=[pl.BlockSpec((1, w), index_map=lambda i: (0, i))],
            out_specs=[pl.BlockSpec((w, d), index_map=lambda i: (i, 0))],
            core_axis_name='subcore',
            dimension_semantics=(pltpu.PARALLEL,),
        )(i_hbm, o_hbm)

    return k(table, idx.reshape(1, m))


def _mlp(cat, w1, b1, w2, b2, w3, b3):
    h = jnp.maximum(jnp.dot(cat, w1) + b1, 0.0) * _BN_INV
    h = jnp.maximum(jnp.dot(h, w2) + b2, 0.0) * _BN_INV
    h = jnp.maximum(jnp.dot(h, w3) + b3, 0.0) * _BN_INV
    return h


def _edge_mlp_body(xi_ref, xj_ref, w1_ref, b1_ref, w2_ref, b2_ref, w3_ref,
                   b3_ref, msg_ref):
    xi = xi_ref[...]
    xj = xj_ref[...]
    cat = jnp.concatenate([xi, xj - xi], axis=1)
    msg_ref[...] = _mlp(cat, w1_ref[...], b1_ref[...], w2_ref[...], b2_ref[...],
                        w3_ref[...], b3_ref[...])


def _edge_mlp(xi, xj, cp):
    nblk = _E // _BE
    return pl.pallas_call(
        _edge_mlp_body,
        grid=(nblk,),
        in_specs=[pl.BlockSpec((_BE, _F), lambda i: (i, 0)),
                  pl.BlockSpec((_BE, _F), lambda i: (i, 0)),
                  pl.BlockSpec((2 * _F, _HID), lambda i: (0, 0)),
                  pl.BlockSpec((1, _HID), lambda i: (0, 0)),
                  pl.BlockSpec((_HID, _HID), lambda i: (0, 0)),
                  pl.BlockSpec((1, _HID), lambda i: (0, 0)),
                  pl.BlockSpec((_HID, _HID), lambda i: (0, 0)),
                  pl.BlockSpec((1, _HID), lambda i: (0, 0))],
        out_specs=pl.BlockSpec((_BE, _HID), lambda i: (i, 0)),
        out_shape=jax.ShapeDtypeStruct((_E, _HID), jnp.float32),
    )(xi, xj, cp['W1'].T, cp['b1'].reshape(1, -1), cp['W2'].T,
      cp['b2'].reshape(1, -1), cp['W3'].T, cp['b3'].reshape(1, -1))


def _segmax_body(dst_ref, msg_ref, out_ref):
    i = pl.program_id(0)

    @pl.when(i == 0)
    def _():
        out_ref[...] = jnp.full((_N, _HID), -_INF, jnp.float32)

    def eb(e, carry):
        d = dst_ref[0, 0, e]
        row = out_ref[pl.ds(d, 1), :]
        out_ref[pl.ds(d, 1), :] = jnp.maximum(row, msg_ref[pl.ds(e, 1), :])
        return carry

    lax.fori_loop(0, _BE, eb, 0)


def _segmax(msg, dst3d):
    nblk = _E // _BE
    return pl.pallas_call(
        _segmax_body,
        grid=(nblk,),
        in_specs=[pl.BlockSpec((1, 1, _BE), lambda i: (i, 0, 0),
                               memory_space=pltpu.SMEM),
                  pl.BlockSpec((_BE, _HID), lambda i: (i, 0))],
        out_specs=pl.BlockSpec((_N, _HID), lambda i: (0, 0)),
        out_shape=jax.ShapeDtypeStruct((_N, _HID), jnp.float32),
    )(dst3d, msg)


def _knn_body(lo_ref, hi_ref, xr_ref, batr_ref, xpad_ref, bat2d_ref, nbr_ref):
    i = pl.program_id(0)
    xr = xr_ref[...]
    batr = batr_ref[...]
    rid = i * _R + lax.broadcasted_iota(jnp.int32, (_R, 1), 0)
    sq_r = jnp.sum(xr * xr, axis=1, keepdims=True)
    aug_r = jnp.concatenate([-2.0 * xr, jnp.ones((_R, 1), jnp.float32)], axis=1)

    lo = lo_ref[i, 0]
    hi = hi_ref[i, 0]
    cb0 = lo // _R
    cb1 = (hi + _R - 1) // _R

    colio = lax.broadcasted_iota(jnp.int32, (_R, _R), 1)
    cio8 = lax.broadcasted_iota(jnp.int32, (_R, 2 * _K), 1)

    def chunk(cb, carry):
        td, ti = carry
        c0 = cb * _R
        xc = xpad_ref[pl.ds(c0, _R), :]
        batc = bat2d_ref[pl.ds(cb, 1), :]
        sq_c = jnp.sum(xc * xc, axis=1, keepdims=True)
        aug_c = jnp.concatenate([xc, sq_c], axis=1)
        d = sq_r + lax.dot_general(aug_r, aug_c, (((1,), (1,)), ((), ())))
        d = jnp.where(batr != batc, _INF, d)
        cid = c0 + lax.broadcasted_iota(jnp.int32, (1, _R), 1)
        d = jnp.where(rid == cid, _INF, d)
        nd, ni = [], []
        for _ in range(_K):
            m = jnp.min(d, axis=1, keepdims=True)
            am = jnp.min(jnp.where(d == m, colio, 2 ** 30), axis=1,
                         keepdims=True)
            nd.append(m)
            ni.append(am + c0)
            d = jnp.where(colio == am, _INF, d)
        cd = jnp.concatenate([td] + nd, axis=1)
        ci = jnp.concatenate([ti] + ni, axis=1)
        ntd, nti = [], []
        for _ in range(_K):
            m = jnp.min(cd, axis=1, keepdims=True)
            pos = jnp.min(jnp.where(cd == m, cio8, 2 ** 30), axis=1,
                          keepdims=True)
            idx = jnp.sum(jnp.where(cio8 == pos, ci, 0), axis=1, keepdims=True)
            ntd.append(m)
            nti.append(idx)
            cd = jnp.where(cio8 == pos, _INF, cd)
        return (jnp.concatenate(ntd, axis=1), jnp.concatenate(nti, axis=1))

    td0 = jnp.full((_R, _K), _INF, jnp.float32)
    ti0 = jnp.zeros((_R, _K), jnp.int32)
    _, ti = lax.fori_loop(cb0, cb1, chunk, (td0, ti0))
    nbr_ref[...] = ti


def _knn(xpad, batf_pad, bat2d, lo, hi):
    return pl.pallas_call(
        _knn_body,
        grid=(_NBLK,),
        in_specs=[pl.BlockSpec((_NBLK, 1), lambda i: (0, 0),
                               memory_space=pltpu.SMEM),
                  pl.BlockSpec((_NBLK, 1), lambda i: (0, 0),
                               memory_space=pltpu.SMEM),
                  pl.BlockSpec((_R, _F), lambda i: (i, 0)),
                  pl.BlockSpec((_R, 1), lambda i: (i, 0)),
                  pl.BlockSpec((_NPAD, _F), lambda i: (0, 0)),
                  pl.BlockSpec((_NBLK, _R), lambda i: (0, 0))],
        out_specs=pl.BlockSpec((_R, _K), lambda i: (i, 0)),
        out_shape=jax.ShapeDtypeStruct((_NPAD, _K), jnp.int32),
    )(lo, hi, xpad, batf_pad, xpad, bat2d)


def _knn_conv_body(x_ref, xj_ref, w1_ref, b1_ref, w2_ref, b2_ref, w3_ref,
                   b3_ref, wst_ref, bs_ref, o_ref):
    xi = x_ref[...]
    xj = xj_ref[...]
    xi4 = jnp.reshape(
        jnp.broadcast_to(xi[:, None, :], (_BN, _K, _HID)), (_BN * _K, _HID))
    cat = jnp.concatenate([xi4, xj - xi4], axis=1)
    h = _mlp(cat, w1_ref[...], b1_ref[...], w2_ref[...], b2_ref[...],
             w3_ref[...], b3_ref[...])
    agg = jnp.max(jnp.reshape(h, (_BN, _K, _HID)), axis=1)
    short = (jnp.dot(xi, wst_ref[...]) + bs_ref[...]) * _BN_INV
    o_ref[...] = agg + short


def _knn_conv(x, xj, cp):
    nblk = _N // _BN
    return pl.pallas_call(
        _knn_conv_body,
        grid=(nblk,),
        in_specs=[pl.BlockSpec((_BN, _F), lambda i: (i, 0)),
                  pl.BlockSpec((_K * _BN, _F), lambda i: (i, 0)),
                  pl.BlockSpec((2 * _F, _HID), lambda i: (0, 0)),
                  pl.BlockSpec((1, _HID), lambda i: (0, 0)),
                  pl.BlockSpec((_HID, _HID), lambda i: (0, 0)),
                  pl.BlockSpec((1, _HID), lambda i: (0, 0)),
                  pl.BlockSpec((_HID, _HID), lambda i: (0, 0)),
                  pl.BlockSpec((1, _HID), lambda i: (0, 0)),
                  pl.BlockSpec((_HID, _HID), lambda i: (0, 0)),
                  pl.BlockSpec((1, _HID), lambda i: (0, 0))],
        out_specs=pl.BlockSpec((_BN, _HID), lambda i: (i, 0)),
        out_shape=jax.ShapeDtypeStruct((_N, _HID), jnp.float32),
    )(x, xj, cp['W1'].T, cp['b1'].reshape(1, -1), cp['W2'].T,
      cp['b2'].reshape(1, -1), cp['W3'].T, cp['b3'].reshape(1, -1),
      cp['Ws'].T, cp['bs'].reshape(1, -1))


def _head_body(c1_ref, c2_ref, c3_ref, batf_ref, gi_ref, w1_ref, b1_ref,
               w2_ref, b2_ref, wo_ref, bo_ref, o_ref):
    xs = c1_ref[...] + c2_ref[...] + c3_ref[...]
    batf = batf_ref[...]
    oh = _onehot(batf, _N)
    ones_col = jnp.ones((_N, 1), jnp.float32)
    cnt_col = jnp.maximum(
        lax.dot_general(oh, ones_col, (((0,), (0,)), ((), ()))), 1.0)
    pooled = lax.dot_general(oh, xs, (((0,), (0,)), ((), ()))) / cnt_col
    h = jnp.concatenate([pooled[0:_B, :], gi_ref[...]], axis=1) * _BN_INV
    h = jnp.maximum(jnp.dot(h, w1_ref[...]) + b1_ref[...], 0.0) * _BN_INV
    h = jnp.maximum(jnp.dot(h, w2_ref[...]) + b2_ref[...], 0.0) * _BN_INV
    o_ref[...] = jnp.dot(h, wo_ref[...]) + bo_ref[...]


def _head(c1, c2, c3, batf, gi, params):
    return pl.pallas_call(
        _head_body,
        out_shape=jax.ShapeDtypeStruct((_B, _NC), jnp.float32),
    )(c1, c2, c3, batf, gi, params['dense1']['W'].T,
      params['dense1']['b'].reshape(1, -1), params['dense2']['W'].T,
      params['dense2']['b'].reshape(1, -1), params['out']['W'].T,
      params['out']['b'].reshape(1, -1))


def _dyn_knn_layer(x, cp, batf_pad, bat2d, lo, hi):
    xpad = jnp.pad(x, ((0, _NPAD - _N), (0, 0)))
    nbr = _knn(xpad, batf_pad, bat2d, lo, hi)
    flat = nbr[:_N].reshape(-1)
    flat = jnp.concatenate([flat, jnp.zeros((64,), jnp.int32)])
    xj = _gather_rows(x, flat)
    return _knn_conv(x, xj[:_K * _N], cp)


def kernel(x, edge_index, graph_input, batch, params):
    batch = batch.astype(jnp.int32)
    edge_index = edge_index.astype(jnp.int32)
    batf = batch.astype(jnp.float32).reshape(_N, 1)
    batch_pad = jnp.concatenate(
        [batch, jnp.full((_NPAD - _N,), _B, jnp.int32)])
    batf_pad = batch_pad.astype(jnp.float32).reshape(_NPAD, 1)
    bat2d = batch_pad.astype(jnp.float32).reshape(_NBLK, _R)
    lo, hi = _block_bounds(batf_pad)

    x0 = _gn_plain(x, batf, params['gn0'])

    dst = edge_index[1]
    src = edge_index[0]
    xi = _gather_rows(x0, dst)
    xj = _gather_rows(x0, src)
    msg = _edge_mlp(xi, xj, params['c1'])
    agg = _segmax(msg, dst.reshape(_E // _BE, 1, _BE))
    c1 = _gn_conv(agg, x0, params['c1'], batf, params['gn1'])

    conv2 = _dyn_knn_layer(c1, params['c2'], batf_pad, bat2d, lo, hi)
    c2 = _gn_plain(conv2, batf, params['gn2'])
    conv3 = _dyn_knn_layer(c2, params['c3'], batf_pad, bat2d, lo, hi)
    c3 = _gn_plain(conv3, batf, params['gn3'])

    return _head(c1, c2, c3, batf, graph_input, params)


# speedup vs baseline: 8.4423x; 1.3273x over previous
import functools
import math

import jax
import jax.numpy as jnp
from jax import lax
from jax.experimental import pallas as pl
from jax.experimental.pallas import tpu as pltpu
from jax.experimental.pallas import tpu_sc as plsc

_N = 10000
_E = 160000
_F = 128
_HID = 128
_GF = 16
_NC = 10
_B = 64
_K = 4
_BN_INV = 1.0 / math.sqrt(1.0 + 1e-5)
_GN_EPS = 1e-5

_R = 256
_NPAD = 10240
_NBLK = _NPAD // _R
_BE = 2000
_BN = 1000
_INF = jnp.inf


def _onehot(batf, n):
    io = lax.broadcasted_iota(jnp.int32, (n, 128), 1).astype(jnp.float32)
    return jnp.where(batf == io, 1.0, 0.0)


def _bounds_body(batf_ref, lo_ref, hi_ref):
    batf = batf_ref[...]
    oh = _onehot(batf, _NPAD)
    ones_col = jnp.ones((_NPAD, 1), jnp.float32)
    cnt_col = lax.dot_general(oh, ones_col, (((0,), (0,)), ((), ())))
    li = lax.broadcasted_iota(jnp.int32, (128, 128), 0)
    lj = lax.broadcasted_iota(jnp.int32, (128, 128), 1)
    lower = jnp.where(li < lj, 1.0, 0.0)
    starts_col = lax.dot_general(lower, cnt_col, (((0,), (0,)), ((), ())))
    ends_col = starts_col + cnt_col
    rio = lax.broadcasted_iota(jnp.int32, (_NBLK, _NPAD), 1)
    bio = lax.broadcasted_iota(jnp.int32, (_NBLK, _NPAD), 0)
    sel_first = jnp.where(rio == bio * _R, 1.0, 0.0)
    sel_last = jnp.where(rio == bio * _R + (_R - 1), 1.0, 0.0)
    b_first = lax.dot_general(sel_first, batf, (((1,), (0,)), ((), ())))
    b_last = lax.dot_general(sel_last, batf, (((1,), (0,)), ((), ())))
    oh_f = _onehot(b_first, _NBLK)
    oh_l = _onehot(b_last, _NBLK)
    lo = lax.dot_general(oh_f, starts_col, (((1,), (0,)), ((), ())))
    hi = lax.dot_general(oh_l, ends_col, (((1,), (0,)), ((), ())))
    lo_ref[...] = lo.astype(jnp.int32)
    hi_ref[...] = hi.astype(jnp.int32)


def _block_bounds(batf_pad):
    return pl.pallas_call(
        _bounds_body,
        out_shape=[jax.ShapeDtypeStruct((_NBLK, 1), jnp.int32),
                   jax.ShapeDtypeStruct((_NBLK, 1), jnp.int32)],
    )(batf_pad)


def _gn_math(t, batf, w, b, ms):
    oh = _onehot(batf, _N)
    ones_col = jnp.ones((_N, 1), jnp.float32)
    cnt_col = lax.dot_general(oh, ones_col, (((0,), (0,)), ((), ())))
    cnt_col = jnp.maximum(cnt_col, 1.0)
    seg_sum = lax.dot_general(oh, t, (((0,), (0,)), ((), ())))
    mean = seg_sum / cnt_col
    mean_b = jnp.dot(oh, mean)
    out = t - mean_b * ms
    var_sum = lax.dot_general(oh, out * out, (((0,), (0,)), ((), ())))
    std = jnp.sqrt(var_sum / cnt_col + _GN_EPS)
    std_b = jnp.dot(oh, std)
    return w * out / std_b + b


def _gn_plain_body(t_ref, batf_ref, w_ref, b_ref, ms_ref, o_ref):
    o_ref[...] = _gn_math(t_ref[...], batf_ref[...], w_ref[...], b_ref[...],
                          ms_ref[...])


def _gn_plain(t, batf, p):
    return pl.pallas_call(
        _gn_plain_body,
        out_shape=jax.ShapeDtypeStruct((_N, _HID), jnp.float32),
    )(t, batf, p['w'].reshape(1, -1), p['b'].reshape(1, -1),
      p['ms'].reshape(1, -1))


def _gn_conv_body(agg_ref, xp_ref, wst_ref, bs_ref, batf_ref, w_ref, b_ref,
                  ms_ref, o_ref):
    agg = agg_ref[...]
    agg = jnp.where(agg == -_INF, 0.0, agg)
    short = (jnp.dot(xp_ref[...], wst_ref[...]) + bs_ref[...]) * _BN_INV
    o_ref[...] = _gn_math(agg + short, batf_ref[...], w_ref[...], b_ref[...],
                          ms_ref[...])


def _gn_conv(agg, xprev, cp, batf, gp):
    return pl.pallas_call(
        _gn_conv_body,
        out_shape=jax.ShapeDtypeStruct((_N, _HID), jnp.float32),
    )(agg, xprev, cp['Ws'].T, cp['bs'].reshape(1, -1), batf,
      gp['w'].reshape(1, -1), gp['b'].reshape(1, -1), gp['ms'].reshape(1, -1))


def _gather_rows(table, idx):
    m = idx.shape[0]
    d = table.shape[1]
    w = 128
    mesh = plsc.VectorSubcoreMesh(core_axis_name="core",
                                  subcore_axis_name="subcore")

    @functools.partial(
        pl.kernel,
        out_type=jax.ShapeDtypeStruct((m, d), table.dtype),
        mesh=mesh)
    def k(x_hbm, i_hbm, o_hbm):
        def body(i_vmem, o_vmem):
            pltpu.sync_copy(x_hbm.at[i_vmem.at[0]], o_vmem)

        pltpu.emit_pipeline(
            body,
            grid=(m // w,),
            in_specs=[pl.BlockSpec((1, w), index_map=lambda i: (0, i))],
            out_specs=[pl.BlockSpec((w, d), index_map=lambda i: (i, 0))],
            core_axis_name=('core', 'subcore'),
            dimension_semantics=(pltpu.PARALLEL,),
        )(i_hbm, o_hbm)

    return k(table, idx.reshape(1, m))


def _mlp(cat, w1, b1, w2, b2, w3, b3):
    h = jnp.maximum(jnp.dot(cat, w1) + b1, 0.0) * _BN_INV
    h = jnp.maximum(jnp.dot(h, w2) + b2, 0.0) * _BN_INV
    h = jnp.maximum(jnp.dot(h, w3) + b3, 0.0) * _BN_INV
    return h


def _edge_mlp_body(xi_ref, xj_ref, w1_ref, b1_ref, w2_ref, b2_ref, w3_ref,
                   b3_ref, msg_ref):
    xi = xi_ref[...]
    xj = xj_ref[...]
    cat = jnp.concatenate([xi, xj - xi], axis=1)
    msg_ref[...] = _mlp(cat, w1_ref[...], b1_ref[...], w2_ref[...], b2_ref[...],
                        w3_ref[...], b3_ref[...])


def _edge_mlp(xi, xj, cp):
    nblk = _E // _BE
    return pl.pallas_call(
        _edge_mlp_body,
        grid=(nblk,),
        in_specs=[pl.BlockSpec((_BE, _F), lambda i: (i, 0)),
                  pl.BlockSpec((_BE, _F), lambda i: (i, 0)),
                  pl.BlockSpec((2 * _F, _HID), lambda i: (0, 0)),
                  pl.BlockSpec((1, _HID), lambda i: (0, 0)),
                  pl.BlockSpec((_HID, _HID), lambda i: (0, 0)),
                  pl.BlockSpec((1, _HID), lambda i: (0, 0)),
                  pl.BlockSpec((_HID, _HID), lambda i: (0, 0)),
                  pl.BlockSpec((1, _HID), lambda i: (0, 0))],
        out_specs=pl.BlockSpec((_BE, _HID), lambda i: (i, 0)),
        out_shape=jax.ShapeDtypeStruct((_E, _HID), jnp.float32),
    )(xi, xj, cp['W1'].T, cp['b1'].reshape(1, -1), cp['W2'].T,
      cp['b2'].reshape(1, -1), cp['W3'].T, cp['b3'].reshape(1, -1))


def _segmax_body(dst_ref, msg_ref, out_ref):
    i = pl.program_id(0)

    @pl.when(i == 0)
    def _():
        out_ref[...] = jnp.full((_N, _HID), -_INF, jnp.float32)

    def eb(e, carry):
        e0 = 4 * e
        ds = [dst_ref[0, 0, e0 + j] for j in range(4)]
        ms = [msg_ref[pl.ds(e0 + j, 1), :] for j in range(4)]
        rs = [out_ref[pl.ds(ds[j], 1), :] for j in range(4)]
        for k in range(4):
            acc = rs[k]
            for j in range(k):
                acc = jnp.where(ds[j] == ds[k], jnp.maximum(acc, ms[j]), acc)
            out_ref[pl.ds(ds[k], 1), :] = jnp.maximum(acc, ms[k])
        return carry

    lax.fori_loop(0, _BE // 4, eb, 0)


def _segmax(msg, dst3d):
    nblk = _E // _BE
    return pl.pallas_call(
        _segmax_body,
        grid=(nblk,),
        in_specs=[pl.BlockSpec((1, 1, _BE), lambda i: (i, 0, 0),
                               memory_space=pltpu.SMEM),
                  pl.BlockSpec((_BE, _HID), lambda i: (i, 0))],
        out_specs=pl.BlockSpec((_N, _HID), lambda i: (0, 0)),
        out_shape=jax.ShapeDtypeStruct((_N, _HID), jnp.float32),
    )(dst3d, msg)


def _knn_body(lo_ref, hi_ref, xr_ref, batr_ref, xpad_ref, bat2d_ref, nbr_ref):
    i = pl.program_id(0)
    xr = xr_ref[...]
    batr = batr_ref[...]
    rid = i * _R + lax.broadcasted_iota(jnp.int32, (_R, 1), 0)
    sq_r = jnp.sum(xr * xr, axis=1, keepdims=True)
    aug_r = jnp.concatenate([-2.0 * xr, jnp.ones((_R, 1), jnp.float32)], axis=1)

    lo = lo_ref[i, 0]
    hi = hi_ref[i, 0]
    cb0 = lo // _R
    cb1 = (hi + _R - 1) // _R

    colio = lax.broadcasted_iota(jnp.int32, (_R, _R), 1)
    cio8 = lax.broadcasted_iota(jnp.int32, (_R, 2 * _K), 1)

    def chunk(cb, carry):
        td, ti = carry
        c0 = cb * _R
        xc = xpad_ref[pl.ds(c0, _R), :]
        batc = bat2d_ref[pl.ds(cb, 1), :]
        sq_c = jnp.sum(xc * xc, axis=1, keepdims=True)
        aug_c = jnp.concatenate([xc, sq_c], axis=1)
        d = sq_r + lax.dot_general(aug_r, aug_c, (((1,), (1,)), ((), ())))
        d = jnp.where(batr != batc, _INF, d)
        cid = c0 + lax.broadcasted_iota(jnp.int32, (1, _R), 1)
        d = jnp.where(rid == cid, _INF, d)
        nd, ni = [], []
        for _ in range(_K):
            m = jnp.min(d, axis=1, keepdims=True)
            am = jnp.min(jnp.where(d == m, colio, 2 ** 30), axis=1,
                         keepdims=True)
            nd.append(m)
            ni.append(am + c0)
            d = jnp.where(colio == am, _INF, d)
        cd = jnp.concatenate([td] + nd, axis=1)
        ci = jnp.concatenate([ti] + ni, axis=1)
        ntd, nti = [], []
        for _ in range(_K):
            m = jnp.min(cd, axis=1, keepdims=True)
            pos = jnp.min(jnp.where(cd == m, cio8, 2 ** 30), axis=1,
                          keepdims=True)
            idx = jnp.sum(jnp.where(cio8 == pos, ci, 0), axis=1, keepdims=True)
            ntd.append(m)
            nti.append(idx)
            cd = jnp.where(cio8 == pos, _INF, cd)
        return (jnp.concatenate(ntd, axis=1), jnp.concatenate(nti, axis=1))

    td0 = jnp.full((_R, _K), _INF, jnp.float32)
    ti0 = jnp.zeros((_R, _K), jnp.int32)
    _, ti = lax.fori_loop(cb0, cb1, chunk, (td0, ti0))
    nbr_ref[...] = ti


def _knn(xpad, batf_pad, bat2d, lo, hi):
    return pl.pallas_call(
        _knn_body,
        grid=(_NBLK,),
        in_specs=[pl.BlockSpec((_NBLK, 1), lambda i: (0, 0),
                               memory_space=pltpu.SMEM),
                  pl.BlockSpec((_NBLK, 1), lambda i: (0, 0),
                               memory_space=pltpu.SMEM),
                  pl.BlockSpec((_R, _F), lambda i: (i, 0)),
                  pl.BlockSpec((_R, 1), lambda i: (i, 0)),
                  pl.BlockSpec((_NPAD, _F), lambda i: (0, 0)),
                  pl.BlockSpec((_NBLK, _R), lambda i: (0, 0))],
        out_specs=pl.BlockSpec((_R, _K), lambda i: (i, 0)),
        out_shape=jax.ShapeDtypeStruct((_NPAD, _K), jnp.int32),
    )(lo, hi, xpad, batf_pad, xpad, bat2d)


def _knn_conv_body(x_ref, xj_ref, w1_ref, b1_ref, w2_ref, b2_ref, w3_ref,
                   b3_ref, wst_ref, bs_ref, o_ref):
    xi = x_ref[...]
    xj = xj_ref[...]
    xi4 = jnp.reshape(
        jnp.broadcast_to(xi[:, None, :], (_BN, _K, _HID)), (_BN * _K, _HID))
    cat = jnp.concatenate([xi4, xj - xi4], axis=1)
    h = _mlp(cat, w1_ref[...], b1_ref[...], w2_ref[...], b2_ref[...],
             w3_ref[...], b3_ref[...])
    agg = jnp.max(jnp.reshape(h, (_BN, _K, _HID)), axis=1)
    short = (jnp.dot(xi, wst_ref[...]) + bs_ref[...]) * _BN_INV
    o_ref[...] = agg + short


def _knn_conv(x, xj, cp):
    nblk = _N // _BN
    return pl.pallas_call(
        _knn_conv_body,
        grid=(nblk,),
        in_specs=[pl.BlockSpec((_BN, _F), lambda i: (i, 0)),
                  pl.BlockSpec((_K * _BN, _F), lambda i: (i, 0)),
                  pl.BlockSpec((2 * _F, _HID), lambda i: (0, 0)),
                  pl.BlockSpec((1, _HID), lambda i: (0, 0)),
                  pl.BlockSpec((_HID, _HID), lambda i: (0, 0)),
                  pl.BlockSpec((1, _HID), lambda i: (0, 0)),
                  pl.BlockSpec((_HID, _HID), lambda i: (0, 0)),
                  pl.BlockSpec((1, _HID), lambda i: (0, 0)),
                  pl.BlockSpec((_HID, _HID), lambda i: (0, 0)),
                  pl.BlockSpec((1, _HID), lambda i: (0, 0))],
        out_specs=pl.BlockSpec((_BN, _HID), lambda i: (i, 0)),
        out_shape=jax.ShapeDtypeStruct((_N, _HID), jnp.float32),
    )(x, xj, cp['W1'].T, cp['b1'].reshape(1, -1), cp['W2'].T,
      cp['b2'].reshape(1, -1), cp['W3'].T, cp['b3'].reshape(1, -1),
      cp['Ws'].T, cp['bs'].reshape(1, -1))


def _head_body(c1_ref, c2_ref, c3_ref, batf_ref, gi_ref, w1_ref, b1_ref,
               w2_ref, b2_ref, wo_ref, bo_ref, o_ref):
    xs = c1_ref[...] + c2_ref[...] + c3_ref[...]
    batf = batf_ref[...]
    oh = _onehot(batf, _N)
    ones_col = jnp.ones((_N, 1), jnp.float32)
    cnt_col = jnp.maximum(
        lax.dot_general(oh, ones_col, (((0,), (0,)), ((), ()))), 1.0)
    pooled = lax.dot_general(oh, xs, (((0,), (0,)), ((), ()))) / cnt_col
    h = jnp.concatenate([pooled[0:_B, :], gi_ref[...]], axis=1) * _BN_INV
    h = jnp.maximum(jnp.dot(h, w1_ref[...]) + b1_ref[...], 0.0) * _BN_INV
    h = jnp.maximum(jnp.dot(h, w2_ref[...]) + b2_ref[...], 0.0) * _BN_INV
    o_ref[...] = jnp.dot(h, wo_ref[...]) + bo_ref[...]


def _head(c1, c2, c3, batf, gi, params):
    return pl.pallas_call(
        _head_body,
        out_shape=jax.ShapeDtypeStruct((_B, _NC), jnp.float32),
    )(c1, c2, c3, batf, gi, params['dense1']['W'].T,
      params['dense1']['b'].reshape(1, -1), params['dense2']['W'].T,
      params['dense2']['b'].reshape(1, -1), params['out']['W'].T,
      params['out']['b'].reshape(1, -1))


def _dyn_knn_layer(x, cp, batf_pad, bat2d, lo, hi):
    xpad = jnp.pad(x, ((0, _NPAD - _N), (0, 0)))
    nbr = _knn(xpad, batf_pad, bat2d, lo, hi)
    flat = nbr[:_N].reshape(-1)
    flat = jnp.concatenate([flat, jnp.zeros((64,), jnp.int32)])
    xj = _gather_rows(x, flat)
    return _knn_conv(x, xj[:_K * _N], cp)


def kernel(x, edge_index, graph_input, batch, params):
    batch = batch.astype(jnp.int32)
    edge_index = edge_index.astype(jnp.int32)
    batf = batch.astype(jnp.float32).reshape(_N, 1)
    batch_pad = jnp.concatenate(
        [batch, jnp.full((_NPAD - _N,), _B, jnp.int32)])
    batf_pad = batch_pad.astype(jnp.float32).reshape(_NPAD, 1)
    bat2d = batch_pad.astype(jnp.float32).reshape(_NBLK, _R)
    lo, hi = _block_bounds(batf_pad)

    x0 = _gn_plain(x, batf, params['gn0'])

    dst = edge_index[1]
    src = edge_index[0]
    xi = _gather_rows(x0, dst)
    xj = _gather_rows(x0, src)
    msg = _edge_mlp(xi, xj, params['c1'])
    agg = _segmax(msg, dst.reshape(_E // _BE, 1, _BE))
    c1 = _gn_conv(agg, x0, params['c1'], batf, params['gn1'])

    conv2 = _dyn_knn_layer(c1, params['c2'], batf_pad, bat2d, lo, hi)
    c2 = _gn_plain(conv2, batf, params['gn2'])
    conv3 = _dyn_knn_layer(c2, params['c3'], batf_pad, bat2d, lo, hi)
    c3 = _gn_plain(conv3, batf, params['gn3'])

    return _head(c1, c2, c3, batf, graph_input, params)
